# baseline (speedup 1.0000x reference)
import functools

import jax
import jax.numpy as jnp
from jax import lax
from jax.experimental import pallas as pl
from jax.experimental.pallas import tpu as pltpu
from jax.experimental.pallas import tpu_sc as plsc

BATCH = 16384
EMBED_DIM = 32
VOCAB_GROUPS = 250000

_NUM_CORES = 2
_NUM_SUBCORES = 16
_NUM_WORKERS = _NUM_CORES * _NUM_SUBCORES
_B_PER_W = BATCH // _NUM_WORKERS
_CHUNK = 128
_N_CHUNKS = _B_PER_W // _CHUNK
_LANES = 16


@functools.partial(
    pl.kernel,
    out_type=jax.ShapeDtypeStruct((BATCH, 128), jnp.float32),
    mesh=plsc.VectorSubcoreMesh(core_axis_name="c", subcore_axis_name="s"),
    scratch_types=[
        pltpu.VMEM((_B_PER_W,), jnp.int32),
        pltpu.VMEM((_B_PER_W,), jnp.int32),
        pltpu.VMEM((_B_PER_W, 128), jnp.float32),
        pltpu.SemaphoreType.DMA,
    ],
)
def _gather_kernel(q_hbm, r_hbm, table4_hbm, out_hbm, q_v, r_v, raw_v, sem):
    wid = lax.axis_index("s") * _NUM_CORES + lax.axis_index("c")
    base = wid * _B_PER_W
    pltpu.sync_copy(q_hbm.at[wid], q_v)
    pltpu.sync_copy(r_hbm.at[wid], r_v)

    copies = [
        pltpu.async_copy(
            table4_hbm.at[q_v.at[pl.ds(j * _CHUNK, _CHUNK)]],
            raw_v.at[pl.ds(j * _CHUNK, _CHUNK)],
            sem,
        )
        for j in range(_N_CHUNKS)
    ]
    for c in copies:
        c.wait()

    def extract(g, _):
        row0 = g * _LANES
        rvec = r_v[pl.ds(row0, _LANES)]
        one = jnp.full((_LANES,), 1, jnp.int32)
        for u in range(_LANES):
            i = row0 + u
            pv = jnp.full((_LANES,), rvec[u], jnp.int32)
            masks = [
                (one - jnp.minimum(jnp.abs(pv - k), one)).astype(jnp.float32)
                for k in range(4)
            ]
            for h in range(2):
                acc = raw_v[i, pl.ds(h * _LANES, _LANES)] * masks[0]
                for k in range(1, 4):
                    acc = acc + raw_v[
                        i, pl.ds(32 * k + h * _LANES, _LANES)
                    ] * masks[k]
                raw_v[i, pl.ds(h * _LANES, _LANES)] = acc
        return ()

    lax.fori_loop(0, _B_PER_W // _LANES, extract, ())
    pltpu.sync_copy(raw_v, out_hbm.at[pl.ds(base, _B_PER_W)])


def kernel(item_ids, item_embedding):
    ids = item_ids.astype(jnp.int32)
    q = (ids // 4).reshape(_NUM_WORKERS, _B_PER_W)
    r = (ids % 4).reshape(_NUM_WORKERS, _B_PER_W)
    table4 = item_embedding.reshape(VOCAB_GROUPS, 128)
    out_pad = _gather_kernel(q, r, table4)
    return out_pad[:, :EMBED_DIM]


# speedup vs baseline: 1.0352x; 1.0352x over previous
import functools

import jax
import jax.numpy as jnp
from jax import lax
from jax.experimental import pallas as pl
from jax.experimental.pallas import tpu as pltpu
from jax.experimental.pallas import tpu_sc as plsc

BATCH = 16384
EMBED_DIM = 32
VOCAB = 1000000

_NUM_CORES = 2
_NUM_SUBCORES = 16
_NUM_WORKERS = _NUM_CORES * _NUM_SUBCORES
_B_PER_W = BATCH // _NUM_WORKERS
_CHUNK = 128
_N_CHUNKS = _B_PER_W // _CHUNK


@functools.partial(
    pl.kernel,
    out_type=jax.ShapeDtypeStruct((BATCH, 128), jnp.float32),
    mesh=plsc.VectorSubcoreMesh(core_axis_name="c", subcore_axis_name="s"),
    scratch_types=[
        pltpu.VMEM((_B_PER_W,), jnp.int32),
        pltpu.VMEM((_B_PER_W, 128), jnp.float32),
        pltpu.SemaphoreType.DMA,
    ],
)
def _gather_kernel(idx_hbm, table_hbm, out_hbm, idx_v, raw_v, sem):
    wid = lax.axis_index("s") * _NUM_CORES + lax.axis_index("c")
    base = wid * _B_PER_W
    pltpu.sync_copy(idx_hbm.at[wid], idx_v)
    copies = [
        pltpu.async_copy(
            table_hbm.at[idx_v.at[pl.ds(j * _CHUNK, _CHUNK)]],
            raw_v.at[pl.ds(j * _CHUNK, _CHUNK)],
            sem,
        )
        for j in range(_N_CHUNKS)
    ]
    for c in copies:
        c.wait()
    pltpu.sync_copy(raw_v, out_hbm.at[pl.ds(base, _B_PER_W)])


def kernel(item_ids, item_embedding):
    idx = item_ids.astype(jnp.int32).reshape(_NUM_WORKERS, _B_PER_W)
    table_pad = jnp.pad(item_embedding, ((0, 0), (0, 128 - EMBED_DIM)))
    out_pad = _gather_kernel(idx, table_pad)
    return out_pad[:, :EMBED_DIM]
